# baseline (speedup 1.0000x reference)
import functools

import jax
import jax.numpy as jnp
from jax import lax
from jax.experimental import pallas as pl
from jax.experimental.pallas import tpu as pltpu
from jax.experimental.pallas import tpu_sc as plsc

NUM_EMB = 1000000
DIM = 64
SEQ = 16384
NUM_WORKERS = 32
B_PER_W = SEQ // NUM_WORKERS
L = 16


def _body(table_hbm, idx_hbm, out_hbm, idx_v, rows_v, outw_v, sem):
    wid = lax.axis_index("s") * 2 + lax.axis_index("c")
    base = wid * B_PER_W
    pltpu.sync_copy(idx_hbm.at[pl.ds(base, B_PER_W)], idx_v)
    pltpu.async_copy(table_hbm.at[idx_v], rows_v, sem).wait()

    even = lax.iota(jnp.int32, L) * 2
    odd = even + 1

    def widen(i, _):
        for half in range(2):
            x = rows_v[i, pl.ds(half * 2 * L, 2 * L)]
            a, b = plsc.unpack(x, format=plsc.PackFormat.INTERLEAVED)
            plsc.store_scatter(outw_v, [jnp.full((L,), i, jnp.int32),
                                        even + half * 2 * L], a)
            plsc.store_scatter(outw_v, [jnp.full((L,), i, jnp.int32),
                                        odd + half * 2 * L], b)
        return ()

    lax.fori_loop(0, B_PER_W, widen, ())
    pltpu.sync_copy(outw_v, out_hbm.at[pl.ds(base, B_PER_W)])


def kernel(token_ids, embedding_matrix):
    tab16 = embedding_matrix.astype(jnp.bfloat16)
    mesh = plsc.VectorSubcoreMesh(core_axis_name="c", subcore_axis_name="s")
    k = pl.kernel(
        _body,
        mesh=mesh,
        out_type=jax.ShapeDtypeStruct((SEQ, DIM), jnp.float32),
        scratch_types=[
            pltpu.VMEM((B_PER_W,), jnp.int32),
            pltpu.VMEM((B_PER_W, DIM), jnp.bfloat16),
            pltpu.VMEM((B_PER_W, DIM), jnp.float32),
            pltpu.SemaphoreType.DMA,
        ],
        compiler_params=pltpu.CompilerParams(
            use_tc_tiling_on_sc=False, needs_layout_passes=False),
    )
    return k(tab16, token_ids.astype(jnp.int32))


# speedup vs baseline: 2.3048x; 2.3048x over previous
import functools

import jax
import jax.numpy as jnp
from jax import lax
from jax.experimental import pallas as pl
from jax.experimental.pallas import tpu as pltpu
from jax.experimental.pallas import tpu_sc as plsc

NUM_EMB = 1000000
DIM = 64
SEQ = 16384
NUM_WORKERS = 32
B_PER_W = SEQ // NUM_WORKERS
FLIGHT = 32
NG = B_PER_W // FLIGHT
DEPTH = 4


def _body(table_hbm, idx_hbm, out_hbm, idx_v, rows_v, sem0, sem1):
    wid = lax.axis_index("s") * 2 + lax.axis_index("c")
    base = wid * B_PER_W
    pltpu.sync_copy(idx_hbm.at[pl.ds(base, B_PER_W)], idx_v)
    sems = (sem0, sem1)

    def fire_s(g, sem):
        gb = g * FLIGHT
        for v16 in range(FLIGHT // 16):
            vec = idx_v[pl.ds(gb + v16 * 16, 16)]
            for i in range(16):
                row = vec[i]
                pltpu.async_copy(
                    table_hbm.at[pl.ds(row, 1), :],
                    rows_v.at[pl.ds(gb + v16 * 16 + i, 1), :],
                    sem,
                )

    def drain_s(g, sem):
        pltpu.make_async_copy(
            table_hbm.at[pl.ds(0, FLIGHT), :],
            rows_v.at[pl.ds(g * FLIGHT, FLIGHT), :],
            sem,
        ).wait()

    for g in range(DEPTH):
        fire_s(g, sems[g % 2])

    def pair(p, _):
        g = 2 * p
        drain_s(g, sems[0])
        fire_s(g + DEPTH, sems[0])
        drain_s(g + 1, sems[1])
        fire_s(g + DEPTH + 1, sems[1])
        return ()

    lax.fori_loop(0, (NG - DEPTH) // 2, pair, ())
    for g in range(NG - DEPTH, NG):
        drain_s(g, sems[g % 2])
    pltpu.sync_copy(rows_v, out_hbm.at[pl.ds(base, B_PER_W)])


def kernel(token_ids, embedding_matrix):
    mesh = plsc.VectorSubcoreMesh(core_axis_name="c", subcore_axis_name="s")
    k = pl.kernel(
        _body,
        mesh=mesh,
        out_type=jax.ShapeDtypeStruct((SEQ, DIM), jnp.float32),
        scratch_types=[
            pltpu.VMEM((B_PER_W,), jnp.int32),
            pltpu.VMEM((B_PER_W, DIM), jnp.float32),
            pltpu.SemaphoreType.DMA,
            pltpu.SemaphoreType.DMA,
        ],
    )
    return k(embedding_matrix, token_ids.astype(jnp.int32))
